# baseline (speedup 1.0000x reference)
import jax
import jax.numpy as jnp
from jax.experimental import pallas as pl
from jax.experimental.pallas import tpu as pltpu


def _h1_kernel(x_ref, w_ref, h_ref):
    h_ref[...] = jnp.dot(
        x_ref[...], w_ref[...], preferred_element_type=jnp.float32
    ).astype(jnp.bfloat16)


def _hq_kernel(x_ref, w_ref, hq_ref, sc_ref):
    h = jnp.dot(x_ref[...], w_ref[...], preferred_element_type=jnp.float32)
    m = jnp.max(jnp.abs(h), axis=0, keepdims=True)
    scale = 127.0 / jnp.maximum(m, 1e-30)
    hq_ref[...] = jnp.round(h * scale).astype(jnp.int8)
    sc_ref[...] = jnp.concatenate(
        [1.0 / (254.0 * scale), 0.5 * jnp.sum(h, axis=0, keepdims=True)],
        axis=0,
    )


def _compute_h1(x, w):
    n, f = x.shape
    h = w.shape[1]
    return pl.pallas_call(
        _h1_kernel,
        out_shape=jax.ShapeDtypeStruct((n, h), jnp.bfloat16),
    )(x, w)


def _compute_hq(x, w):
    n, f = x.shape
    h = w.shape[1]
    return pl.pallas_call(
        _hq_kernel,
        out_shape=[
            jax.ShapeDtypeStruct((n, h), jnp.int8),
            jax.ShapeDtypeStruct((2, h), jnp.float32),
        ],
    )(x, w)


def _stream1_kernel(h_ref, adj_ref, o_ref, adjq_ref):
    a = adj_ref[...]
    adjq_ref[...] = jnp.round((a - 0.5) * 254.0).astype(jnp.int8)
    o_ref[...] = jax.nn.relu(
        jnp.dot(
            a.astype(jnp.bfloat16), h_ref[...],
            preferred_element_type=jnp.float32,
        )
    )


def _streamq_kernel(hq_ref, sc_ref, adjq_ref, o_ref):
    acc = jnp.dot(
        adjq_ref[...], hq_ref[...], preferred_element_type=jnp.int32
    )
    o_ref[...] = jax.nn.relu(
        acc.astype(jnp.float32) * sc_ref[0:1, :] + sc_ref[1:2, :]
    )


def _gcn_layer1(x, adj, w, blk):
    n, f = x.shape
    h = w.shape[1]
    h1 = _compute_h1(x, w)
    return pl.pallas_call(
        _stream1_kernel,
        grid=(n // blk,),
        in_specs=[
            pl.BlockSpec((n, h), lambda i: (0, 0)),
            pl.BlockSpec((blk, n), lambda i: (i, 0)),
        ],
        out_specs=[
            pl.BlockSpec((blk, h), lambda i: (i, 0)),
            pl.BlockSpec((blk, n), lambda i: (i, 0)),
        ],
        out_shape=[
            jax.ShapeDtypeStruct((n, h), jnp.float32),
            jax.ShapeDtypeStruct((n, n), jnp.int8),
        ],
        compiler_params=pltpu.CompilerParams(
            dimension_semantics=("parallel",)
        ),
    )(h1, adj)


def _gcn_layer_q(x, adjq, w, blk):
    n = x.shape[0]
    h = w.shape[1]
    hq, sc = _compute_hq(x, w)
    return pl.pallas_call(
        _streamq_kernel,
        grid=(n // blk,),
        in_specs=[
            pl.BlockSpec((n, h), lambda i: (0, 0)),
            pl.BlockSpec((2, h), lambda i: (0, 0)),
            pl.BlockSpec((blk, n), lambda i: (i, 0)),
        ],
        out_specs=pl.BlockSpec((blk, h), lambda i: (i, 0)),
        out_shape=jax.ShapeDtypeStruct((n, h), jnp.float32),
        compiler_params=pltpu.CompilerParams(
            dimension_semantics=("parallel",)
        ),
    )(hq, sc, adjq)


def kernel(features, adj_matrix, W_in, W_h0, W_out):
    x, adjq = _gcn_layer1(features, adj_matrix, W_in, 400)
    x = _gcn_layer_q(x, adjq, W_h0, 400)
    return _gcn_layer_q(x, adjq, W_out, 400)


# speedup vs baseline: 1.0314x; 1.0314x over previous
import jax
import jax.numpy as jnp
from jax.experimental import pallas as pl
from jax.experimental.pallas import tpu as pltpu


def _layer1_kernel(x_ref, w_ref, adj_ref, o_ref, adjq_ref, h_ref):
    @pl.when(pl.program_id(0) == 0)
    def _():
        h_ref[...] = jnp.dot(
            x_ref[...], w_ref[...], preferred_element_type=jnp.float32
        ).astype(jnp.bfloat16)

    a = adj_ref[...]
    adjq_ref[...] = jnp.round((a - 0.5) * 254.0).astype(jnp.int8)
    o_ref[...] = jax.nn.relu(
        jnp.dot(
            a.astype(jnp.bfloat16), h_ref[...],
            preferred_element_type=jnp.float32,
        )
    )


def _layer_q_kernel(x_ref, w_ref, adjq_ref, o_ref, h_ref, c_ref):
    @pl.when(pl.program_id(0) == 0)
    def _():
        h = jnp.dot(x_ref[...], w_ref[...], preferred_element_type=jnp.float32)
        h_ref[...] = h.astype(jnp.bfloat16)
        c_ref[...] = 0.5 * jnp.sum(h, axis=0, keepdims=True)

    acc = jnp.dot(
        adjq_ref[...], h_ref[...], preferred_element_type=jnp.float32
    )
    o_ref[...] = jax.nn.relu(acc * (1.0 / 254.0) + c_ref[...])


def _gcn_layer1(x, adj, w, blk):
    n, f = x.shape
    h = w.shape[1]
    return pl.pallas_call(
        _layer1_kernel,
        grid=(n // blk,),
        in_specs=[
            pl.BlockSpec((n, f), lambda i: (0, 0)),
            pl.BlockSpec((f, h), lambda i: (0, 0)),
            pl.BlockSpec((blk, n), lambda i: (i, 0)),
        ],
        out_specs=[
            pl.BlockSpec((blk, h), lambda i: (i, 0)),
            pl.BlockSpec((blk, n), lambda i: (i, 0)),
        ],
        out_shape=[
            jax.ShapeDtypeStruct((n, h), jnp.float32),
            jax.ShapeDtypeStruct((n, n), jnp.int8),
        ],
        scratch_shapes=[pltpu.VMEM((n, h), jnp.bfloat16)],
    )(x, w, adj)


def _gcn_layer_q(x, adjq, w, blk):
    n, f = x.shape
    h = w.shape[1]
    return pl.pallas_call(
        _layer_q_kernel,
        grid=(n // blk,),
        in_specs=[
            pl.BlockSpec((n, f), lambda i: (0, 0)),
            pl.BlockSpec((f, h), lambda i: (0, 0)),
            pl.BlockSpec((blk, n), lambda i: (i, 0)),
        ],
        out_specs=pl.BlockSpec((blk, h), lambda i: (i, 0)),
        out_shape=jax.ShapeDtypeStruct((n, h), jnp.float32),
        scratch_shapes=[
            pltpu.VMEM((n, h), jnp.bfloat16),
            pltpu.VMEM((1, h), jnp.float32),
        ],
    )(x, w, adjq)


def kernel(features, adj_matrix, W_in, W_h0, W_out):
    x, adjq = _gcn_layer1(features, adj_matrix, W_in, 400)
    x = _gcn_layer_q(x, adjq, W_h0, 400)
    return _gcn_layer_q(x, adjq, W_out, 400)


# speedup vs baseline: 1.0429x; 1.0111x over previous
import jax
import jax.numpy as jnp
from jax.experimental import pallas as pl
from jax.experimental.pallas import tpu as pltpu


def _layer1_kernel(x_ref, w_ref, adj_ref, o_ref, adjq_ref, h_ref):
    @pl.when(pl.program_id(0) == 0)
    def _():
        h_ref[...] = jnp.dot(
            x_ref[...], w_ref[...], preferred_element_type=jnp.float32
        ).astype(jnp.bfloat16)

    a = adj_ref[...]
    adjq_ref[...] = jnp.round((a - 0.5) * 254.0).astype(jnp.int8)
    o_ref[...] = jax.nn.relu(
        jnp.dot(
            a.astype(jnp.bfloat16), h_ref[...],
            preferred_element_type=jnp.float32,
        )
    )


def _layer_q_kernel(x_ref, w_ref, adjq_ref, o_ref, h_ref, c_ref):
    @pl.when(pl.program_id(0) == 0)
    def _():
        h = jnp.dot(x_ref[...], w_ref[...], preferred_element_type=jnp.float32)
        h_ref[...] = h.astype(jnp.bfloat16)
        c_ref[...] = 0.5 * jnp.sum(h, axis=0, keepdims=True)

    acc = jnp.dot(
        adjq_ref[...], h_ref[...], preferred_element_type=jnp.float32
    )
    o_ref[...] = jax.nn.relu(acc * (1.0 / 254.0) + c_ref[...])


def _gcn_layer1(x, adj, w, blk):
    n, f = x.shape
    h = w.shape[1]
    return pl.pallas_call(
        _layer1_kernel,
        grid=(n // blk,),
        in_specs=[
            pl.BlockSpec((n, f), lambda i: (0, 0)),
            pl.BlockSpec((f, h), lambda i: (0, 0)),
            pl.BlockSpec((blk, n), lambda i: (i, 0)),
        ],
        out_specs=[
            pl.BlockSpec((blk, h), lambda i: (i, 0)),
            pl.BlockSpec((blk, n), lambda i: (i, 0)),
        ],
        out_shape=[
            jax.ShapeDtypeStruct((n, h), jnp.float32),
            jax.ShapeDtypeStruct((n, n), jnp.int8),
        ],
        scratch_shapes=[pltpu.VMEM((n, h), jnp.bfloat16)],
    )(x, w, adj)


def _gcn_layer_q(x, adjq, w, blk):
    n, f = x.shape
    h = w.shape[1]
    return pl.pallas_call(
        _layer_q_kernel,
        grid=(n // blk,),
        in_specs=[
            pl.BlockSpec((n, f), lambda i: (0, 0)),
            pl.BlockSpec((f, h), lambda i: (0, 0)),
            pl.BlockSpec((blk, n), lambda i: (i, 0)),
        ],
        out_specs=pl.BlockSpec((blk, h), lambda i: (i, 0)),
        out_shape=jax.ShapeDtypeStruct((n, h), jnp.float32),
        scratch_shapes=[
            pltpu.VMEM((n, h), jnp.bfloat16),
            pltpu.VMEM((1, h), jnp.float32),
        ],
    )(x, w, adjq)


def kernel(features, adj_matrix, W_in, W_h0, W_out):
    x, adjq = _gcn_layer1(features, adj_matrix, W_in, 400)
    x = _gcn_layer_q(x, adjq, W_h0, 1000)
    return _gcn_layer_q(x, adjq, W_out, 1000)
